# baseline (speedup 1.0000x reference)
import functools

import jax
import jax.numpy as jnp
from jax.experimental import pallas as pl
from jax.experimental.pallas import tpu as pltpu

EPS = 1e-5


def _sage_kernel(a0_ref, a1_ref, x_ref, g_ref, beta_ref, w1_ref, w2_ref, b_ref,
                 o_ref, h_ref, *, bm):
    i = pl.program_id(0)

    @pl.when(i == 0)
    def _compute_ln():
        x = x_ref[...]
        mu = jnp.mean(x, axis=-1, keepdims=True)
        var = jnp.mean((x - mu) * (x - mu), axis=-1, keepdims=True)
        h_ref[...] = (x - mu) * jax.lax.rsqrt(var + EPS) * g_ref[...] + beta_ref[...]

    h = h_ref[...]
    dn = (((1,), (1,)), ((), ()))
    for half, a_ref in enumerate((a0_ref, a1_ref)):
        neigh = jnp.dot(a_ref[...], h, preferred_element_type=jnp.float32)
        base = i * bm + half * (bm // 2)
        h_blk = h_ref[pl.ds(base, bm // 2), :]
        x_blk = x_ref[pl.ds(base, bm // 2), :]
        out = (
            jax.lax.dot_general(h_blk, w1_ref[...], dn, preferred_element_type=jnp.float32)
            + jax.lax.dot_general(neigh, w2_ref[...], dn, preferred_element_type=jnp.float32)
            + b_ref[...]
        )
        o_ref[half * (bm // 2) : (half + 1) * (bm // 2), :] = jnp.maximum(out, 0.0) + x_blk


def kernel(X, A_norm, W, b, ln_gamma, ln_beta):
    N, D = X.shape
    BM = 400
    W1 = W[:, :D]
    W2 = W[:, D:]
    g2 = ln_gamma.reshape(1, D)
    be2 = ln_beta.reshape(1, D)
    b2 = b.reshape(1, -1)

    out = pl.pallas_call(
        functools.partial(_sage_kernel, bm=BM),
        grid=(N // BM,),
        in_specs=[
            pl.BlockSpec((BM // 2, N), lambda i: (2 * i, 0)),
            pl.BlockSpec((BM // 2, N), lambda i: (2 * i + 1, 0)),
            pl.BlockSpec((N, D), lambda i: (0, 0)),
            pl.BlockSpec((1, D), lambda i: (0, 0)),
            pl.BlockSpec((1, D), lambda i: (0, 0)),
            pl.BlockSpec((D, D), lambda i: (0, 0)),
            pl.BlockSpec((D, D), lambda i: (0, 0)),
            pl.BlockSpec((1, W.shape[0]), lambda i: (0, 0)),
        ],
        out_specs=pl.BlockSpec((BM, D), lambda i: (i, 0)),
        out_shape=jax.ShapeDtypeStruct((N, W.shape[0]), jnp.float32),
        scratch_shapes=[pltpu.VMEM((N, D), jnp.float32)],
        compiler_params=pltpu.CompilerParams(dimension_semantics=("arbitrary",)),
    )(A_norm, A_norm, X, g2, be2, W1, W2, b2)
    return out


# speedup vs baseline: 1.1058x; 1.1058x over previous
import functools

import jax
import jax.numpy as jnp
from jax.experimental import pallas as pl
from jax.experimental.pallas import tpu as pltpu

EPS = 1e-5


def _sage_kernel(a_ref, x_ref, g_ref, beta_ref, w1_ref, w2_ref, b_ref, o_ref, h_ref, *, bm):
    i = pl.program_id(0)

    @pl.when(i == 0)
    def _compute_ln():
        x = x_ref[...]
        mu = jnp.mean(x, axis=-1, keepdims=True)
        var = jnp.mean((x - mu) * (x - mu), axis=-1, keepdims=True)
        h_ref[...] = (x - mu) * jax.lax.rsqrt(var + EPS) * g_ref[...] + beta_ref[...]

    neigh = jnp.dot(a_ref[...], h_ref[...], preferred_element_type=jnp.float32)
    h_blk = h_ref[pl.ds(i * bm, bm), :]
    x_blk = x_ref[pl.ds(i * bm, bm), :]
    dn = (((1,), (1,)), ((), ()))
    out = (
        jax.lax.dot_general(h_blk, w1_ref[...], dn, preferred_element_type=jnp.float32)
        + jax.lax.dot_general(neigh, w2_ref[...], dn, preferred_element_type=jnp.float32)
        + b_ref[...]
    )
    o_ref[...] = jnp.maximum(out, 0.0) + x_blk


def kernel(X, A_norm, W, b, ln_gamma, ln_beta):
    N, D = X.shape
    BM = 400
    W1 = W[:, :D]
    W2 = W[:, D:]
    g2 = ln_gamma.reshape(1, D)
    be2 = ln_beta.reshape(1, D)
    b2 = b.reshape(1, -1)

    out = pl.pallas_call(
        functools.partial(_sage_kernel, bm=BM),
        grid=(N // BM,),
        in_specs=[
            pl.BlockSpec((BM, N), lambda i: (i, 0)),
            pl.BlockSpec((N, D), lambda i: (0, 0)),
            pl.BlockSpec((1, D), lambda i: (0, 0)),
            pl.BlockSpec((1, D), lambda i: (0, 0)),
            pl.BlockSpec((D, D), lambda i: (0, 0)),
            pl.BlockSpec((D, D), lambda i: (0, 0)),
            pl.BlockSpec((1, W.shape[0]), lambda i: (0, 0)),
        ],
        out_specs=pl.BlockSpec((BM, D), lambda i: (i, 0)),
        out_shape=jax.ShapeDtypeStruct((N, W.shape[0]), jnp.float32),
        scratch_shapes=[pltpu.VMEM((N, D), jnp.float32)],
        compiler_params=pltpu.CompilerParams(dimension_semantics=("arbitrary",)),
    )(A_norm, X, g2, be2, W1, W2, b2)
    return out
